# baseline (speedup 1.0000x reference)
import functools

import jax
import jax.numpy as jnp
from jax import lax
from jax.experimental import pallas as pl
from jax.experimental.pallas import tpu as pltpu
from jax.experimental.pallas import tpu_sc as plsc

_N = 10000
_E = 320000
_D = 128
_H = 32
_NP = 10240
_NROW = _NP // 128
_NWORK = 32
_CHUNK = 128
_EPW = 10112
_EPAD = _NWORK * _EPW
_NCHUNK = _EPW // _CHUNK
_ROWS_PS = _NP // 16


def _segsum_sc(table, idxp, zrows, width, chunk, k0):
    mesh = plsc.VectorSubcoreMesh(core_axis_name="c", subcore_axis_name="s")
    tot = _EPAD // chunk // 16
    k1 = tot - k0

    @functools.partial(
        pl.kernel,
        out_type=jax.ShapeDtypeStruct((2, _NP, width), jnp.float32),
        mesh=mesh,
        compiler_params=pltpu.CompilerParams(use_tc_tiling_on_sc=False),
        scratch_types=[
            pltpu.VMEM((6, 2, chunk), jnp.int32),
            pltpu.VMEM((4, chunk, width), jnp.float32),
            pltpu.VMEM_SHARED((_NP, width), jnp.float32),
            pltpu.SemaphoreType.DMA,
            pltpu.SemaphoreType.DMA,
            pltpu.SemaphoreType.DMA,
        ],
    )
    def k(table_hbm, idx_hbm, zero_hbm, out_hbm, ib, rows, acc, sem_i, sem_g, sem_s):
        c = lax.axis_index("c")
        s = lax.axis_index("s")
        r0 = s * _ROWS_PS
        pltpu.sync_copy(zero_hbm.at[pl.ds(r0, _ROWS_PS)], acc.at[pl.ds(r0, _ROWS_PS)])
        plsc.subcore_barrier()
        bc = jnp.where(c == 0, s * k0, 16 * k0 + s * k1)
        nch = jnp.where(c == 0, k0, k1)

        for j in range(5):
            pltpu.async_copy(idx_hbm.at[bc + j], ib.at[j], sem_i)
        for j in range(3):
            pltpu.make_async_copy(idx_hbm.at[bc + j], ib.at[j], sem_i).wait()
            pltpu.async_copy(table_hbm.at[ib.at[j, 0]], rows.at[j], sem_g)

        def body(k_, carry):
            p4 = lax.rem(k_, 4)
            i6 = lax.rem(k_, 6)
            pltpu.make_async_copy(table_hbm.at[ib.at[i6, 0]], rows.at[p4], sem_g).wait()
            pltpu.async_copy(rows.at[p4], acc.at[ib.at[i6, 1]], sem_s, add=True)

            @pl.when(k_ >= 1)
            def _():
                p4m = lax.rem(k_ + 3, 4)
                i6m = lax.rem(k_ + 5, 6)
                pltpu.make_async_copy(rows.at[p4m], acc.at[ib.at[i6m, 1]], sem_s).wait()

            @pl.when(k_ + 5 < nch)
            def _():
                i6p5 = lax.rem(k_ + 5, 6)
                pltpu.async_copy(idx_hbm.at[bc + k_ + 5], ib.at[i6p5], sem_i)

            @pl.when(k_ + 3 < nch)
            def _():
                p4p = lax.rem(k_ + 3, 4)
                i6p = lax.rem(k_ + 3, 6)
                pltpu.make_async_copy(idx_hbm.at[bc + k_ + 3], ib.at[i6p], sem_i).wait()
                pltpu.async_copy(table_hbm.at[ib.at[i6p, 0]], rows.at[p4p], sem_g)

            return carry

        lax.fori_loop(0, nch, body, 0)
        pltpu.make_async_copy(
            rows.at[lax.rem(nch - 1, 4)],
            acc.at[ib.at[lax.rem(nch - 1, 6), 1]], sem_s).wait()
        plsc.subcore_barrier()
        pltpu.sync_copy(acc.at[pl.ds(r0, _ROWS_PS)], out_hbm.at[c, pl.ds(r0, _ROWS_PS)])

    return k(table, idxp, zrows)


def _tc_mid(x, p0, p1, W1a, b1a, W1b, b1b):
    def body(x_ref, p0_ref, p1_ref, w1a_ref, b1a_ref, w1b_ref, b1b_ref, o_ref):
        z = x_ref[...] + (p0_ref[...] + p1_ref[...])
        a = jnp.maximum(jnp.dot(z, w1a_ref[...],
                                preferred_element_type=jnp.float32) + b1a_ref[...], 0.0)
        h = jnp.dot(a, w1b_ref[...], preferred_element_type=jnp.float32) + b1b_ref[...]
        o_ref[...] = jnp.maximum(h, 0.0)

    return pl.pallas_call(
        body,
        out_shape=jax.ShapeDtypeStruct((_NP, _H), jnp.float32),
    )(x, p0, p1, W1a, b1a, W1b, b1b)


def _tc_scores(rh, p0, p1, W2a, b2a, W2b, b2b):
    def body(rh_ref, p0_ref, p1_ref, w2a_ref, b2a_ref, w2b_ref, b2b_ref, o_ref):
        z = rh_ref[...] + (p0_ref[...] + p1_ref[...])
        g = jnp.maximum(jnp.dot(z, w2a_ref[...],
                                preferred_element_type=jnp.float32) + b2a_ref[...], 0.0)
        raw = jnp.dot(g, w2b_ref[...], preferred_element_type=jnp.float32) + b2b_ref[...]
        o_ref[...] = jax.nn.sigmoid(raw)

    return pl.pallas_call(
        body,
        out_shape=jax.ShapeDtypeStruct((_NP, 1), jnp.float32),
    )(rh, p0, p1, W2a, b2a, W2b, b2b)


def _tc_keys(s2, n2, bf2):
    def body(s_ref, n_ref, b_ref, o_ref):
        v = s_ref[...] + n_ref[...]
        row = lax.broadcasted_iota(jnp.int32, (_NROW, 128), 0)
        lane = lax.broadcasted_iota(jnp.int32, (_NROW, 128), 1)
        valid = (row * 128 + lane) < _N
        mn = jnp.min(jnp.where(valid, v, jnp.inf))
        mx = jnp.max(jnp.where(valid, v, -jnp.inf))
        rng = mx - mn
        norm = v - mn
        norm = jnp.where(rng > 0, norm / (rng + 1e-5), norm)
        o_ref[...] = jnp.where(valid, b_ref[...] + norm, 1e9)

    return pl.pallas_call(
        body,
        out_shape=jax.ShapeDtypeStruct((_NROW, 128), jnp.float32),
    )(s2, n2, bf2)


def _tc_ranks_perm(keys2, lo, hi, plo, phi):
    def body(keys_ref, lo_ref, hi_ref, plo_ref, phi_ref, o_ref, ranks_s):
        pid = pl.program_id(0)
        iota_s = lax.broadcasted_iota(jnp.int32, (128, 128), 0)
        iota_l = lax.broadcasted_iota(jnp.int32, (128, 128), 1)
        eye = iota_s == iota_l

        @pl.when(pid < _NROW)
        def _():
            bi = pid
            ki_row = keys_ref[pl.ds(bi, 1), :]
            ki_col = jnp.sum(jnp.where(eye, ki_row, 0.0), axis=1, keepdims=True)
            gi = bi * 128 + lax.broadcasted_iota(jnp.int32, (128, 1), 0)

            def jbody(jc, acc):
                kj = keys_ref[pl.ds(jc, 1), :]
                gj = jc * 128 + lax.broadcasted_iota(jnp.int32, (1, 128), 1)
                lt = kj < ki_col
                tie = (kj == ki_col) & (gj < gi)
                m = jnp.where(lt | tie, 1.0, 0.0)
                return acc + jnp.sum(m, axis=1, keepdims=True)

            acc0 = jnp.full((128, 1), 128.0) * lo_ref[bi].astype(jnp.float32)
            acc = lax.fori_loop(lo_ref[bi], hi_ref[bi], jbody, acc0)
            row = jnp.sum(jnp.where(eye, acc, 0.0), axis=0, keepdims=True)
            ranks_s[pl.ds(bi, 1), :] = row.astype(jnp.int32)

        @pl.when(pid >= _NROW)
        def _():
            bp = pid - _NROW
            prow = bp * 128 + lax.broadcasted_iota(jnp.int32, (1, 128), 1)

            def ibody(bi, acc):
                rrow = ranks_s[pl.ds(bi, 1), :]
                rcol = jnp.sum(jnp.where(eye, rrow, 0), axis=1, keepdims=True)
                gcol = bi * 128 + lax.broadcasted_iota(jnp.int32, (128, 1), 0)
                match = rcol == prow
                return acc + jnp.sum(jnp.where(match, gcol, 0),
                                     axis=0, keepdims=True)

            acc = lax.fori_loop(plo_ref[bp], phi_ref[bp], ibody,
                                jnp.zeros((1, 128), jnp.int32))
            o_ref[...] = acc.reshape(1, 1, 128)

    return pl.pallas_call(
        body,
        grid=(2 * _NROW,),
        in_specs=[
            pl.BlockSpec((_NROW, 128), lambda i: (0, 0)),
            pl.BlockSpec(memory_space=pltpu.SMEM),
            pl.BlockSpec(memory_space=pltpu.SMEM),
            pl.BlockSpec(memory_space=pltpu.SMEM),
            pl.BlockSpec(memory_space=pltpu.SMEM),
        ],
        out_specs=pl.BlockSpec((1, 1, 128),
                               lambda i: (jnp.maximum(i - _NROW, 0), 0, 0)),
        out_shape=jax.ShapeDtypeStruct((_NROW, 1, 128), jnp.int32),
        scratch_shapes=[pltpu.VMEM((_NROW, 128), jnp.int32)],
    )(keys2, lo, hi, plo, phi)


def kernel(x, edge_index, batch, W1a, b1a, W1b, b1b, W2a, b2a, W2b, b2b):
    src = edge_index[0].astype(jnp.int32)
    dst = edge_index[1].astype(jnp.int32)
    srcp = jnp.concatenate([src, jnp.zeros((_EPAD - _E,), jnp.int32)])
    dstp = jnp.concatenate([dst, jnp.full((_EPAD - _E,), _NP - 1, jnp.int32)])
    idxp64 = jnp.stack([srcp.reshape(-1, 64), dstp.reshape(-1, 64)], axis=1)
    idxp = jnp.stack([srcp.reshape(-1, _CHUNK), dstp.reshape(-1, _CHUNK)], axis=1)
    x_pad = jnp.pad(x, ((0, _NP - _N), (0, 0)))
    zrowsD = jnp.zeros((_NP, _D), jnp.float32)
    zrowsH = jnp.zeros((_NP, _H), jnp.float32)

    p1 = _segsum_sc(x_pad, idxp64, zrowsD, _D, 64, 210)
    rh = _tc_mid(x_pad, p1[0], p1[1], W1a, b1a.reshape(1, _H), W1b,
                 b1b.reshape(1, _H))
    p2 = _segsum_sc(rh, idxp, zrowsH, _H, _CHUNK, 91)
    scoresA = _tc_scores(rh, p2[0], p2[1], W2a, b2a.reshape(1, _H), W2b,
                         b2b.reshape(1, 1))

    s2 = scoresA.reshape(_NROW, 128)
    noise = jax.random.uniform(jax.random.key(42), (_N,), dtype=jnp.float32) * 1e-05
    n2 = jnp.pad(noise, (0, _NP - _N)).reshape(_NROW, 128)
    batch_pad = jnp.concatenate([batch.astype(jnp.int32),
                                 jnp.full((_NP - _N,), 127, jnp.int32)])
    bf2 = batch_pad.astype(jnp.float32).reshape(_NROW, 128)
    keys2 = _tc_keys(s2, n2, bf2)

    bp2 = batch_pad.reshape(_NROW, 128)
    bmin, bmax = bp2[:, 0], bp2[:, 127]
    lo = jnp.sum((bmax[None, :] < bmin[:, None]), axis=1).astype(jnp.int32)
    hi = jnp.sum((bmin[None, :] <= bmax[:, None]), axis=1).astype(jnp.int32)
    plo = jnp.sum((hi[None, :] <= jnp.arange(_NROW)[:, None]), axis=1).astype(jnp.int32)
    phi = jnp.sum((lo[None, :] <= jnp.arange(_NROW)[:, None]), axis=1).astype(jnp.int32)
    perm3 = _tc_ranks_perm(keys2, lo, hi, plo, phi)

    perm = perm3.reshape(_NP)[:_N]
    scores = scoresA.reshape(_NP)[:_N]
    return (perm, scores)


# speedup vs baseline: 1.0839x; 1.0839x over previous
import functools

import jax
import jax.numpy as jnp
from jax import lax
from jax.experimental import pallas as pl
from jax.experimental.pallas import tpu as pltpu
from jax.experimental.pallas import tpu_sc as plsc

_N = 10000
_E = 320000
_D = 128
_H = 32
_NP = 10240
_NROW = _NP // 128
_NWORK = 32
_CHUNK = 128
_EPW = 10112
_EPAD = _NWORK * _EPW
_NCHUNK = _EPW // _CHUNK
_ROWS_PS = _NP // 16


def _segsum_sc(table, idxp, zrows, width, k0):
    mesh = plsc.VectorSubcoreMesh(core_axis_name="c", subcore_axis_name="s")
    k1 = 2 * _NCHUNK - k0

    @functools.partial(
        pl.kernel,
        out_type=jax.ShapeDtypeStruct((2, _NP, width), jnp.float32),
        mesh=mesh,
        compiler_params=pltpu.CompilerParams(use_tc_tiling_on_sc=False),
        scratch_types=[
            pltpu.VMEM((3, 2, _CHUNK), jnp.int32),
            pltpu.VMEM((2, _CHUNK, width), jnp.float32),
            pltpu.VMEM_SHARED((_NP, width), jnp.float32),
            pltpu.SemaphoreType.DMA,
            pltpu.SemaphoreType.DMA,
            pltpu.SemaphoreType.DMA,
        ],
    )
    def k(table_hbm, idx_hbm, zero_hbm, out_hbm, ib, rows, acc, sem_i, sem_g, sem_s):
        c = lax.axis_index("c")
        s = lax.axis_index("s")
        r0 = s * _ROWS_PS
        pltpu.sync_copy(zero_hbm.at[pl.ds(r0, _ROWS_PS)], acc.at[pl.ds(r0, _ROWS_PS)])
        plsc.subcore_barrier()
        bc = jnp.where(c == 0, s * k0, 16 * k0 + s * k1)
        nch = jnp.where(c == 0, k0, k1)

        pltpu.async_copy(idx_hbm.at[bc], ib.at[0], sem_i)
        pltpu.make_async_copy(idx_hbm.at[bc], ib.at[0], sem_i).wait()
        pltpu.async_copy(table_hbm.at[ib.at[0, 0]], rows.at[0], sem_g)
        pltpu.async_copy(idx_hbm.at[bc + 1], ib.at[1], sem_i)

        def body(k_, carry):
            p = lax.rem(k_, 2)
            i3 = lax.rem(k_, 3)
            pltpu.make_async_copy(table_hbm.at[ib.at[i3, 0]], rows.at[p], sem_g).wait()
            pltpu.async_copy(rows.at[p], acc.at[ib.at[i3, 1]], sem_s, add=True)

            @pl.when(k_ >= 1)
            def _():
                pn = lax.rem(k_ + 1, 2)
                i3m = lax.rem(k_ + 2, 3)
                pltpu.make_async_copy(rows.at[pn], acc.at[ib.at[i3m, 1]], sem_s).wait()

            @pl.when(k_ + 2 < nch)
            def _():
                i3p2 = lax.rem(k_ + 2, 3)
                pltpu.async_copy(idx_hbm.at[bc + k_ + 2], ib.at[i3p2], sem_i)

            @pl.when(k_ + 1 < nch)
            def _():
                pn = lax.rem(k_ + 1, 2)
                i3p = lax.rem(k_ + 1, 3)
                pltpu.make_async_copy(idx_hbm.at[bc + k_ + 1], ib.at[i3p], sem_i).wait()
                pltpu.async_copy(table_hbm.at[ib.at[i3p, 0]], rows.at[pn], sem_g)

            return carry

        lax.fori_loop(0, nch, body, 0)
        pltpu.make_async_copy(
            rows.at[lax.rem(nch - 1, 2)],
            acc.at[ib.at[lax.rem(nch - 1, 3), 1]], sem_s).wait()
        plsc.subcore_barrier()
        pltpu.sync_copy(acc.at[pl.ds(r0, _ROWS_PS)], out_hbm.at[c, pl.ds(r0, _ROWS_PS)])

    return k(table, idxp, zrows)


def _tc_mid(x, p0, p1, W1a, b1a, W1b, b1b):
    def body(x_ref, p0_ref, p1_ref, w1a_ref, b1a_ref, w1b_ref, b1b_ref, o_ref):
        z = x_ref[...] + (p0_ref[...] + p1_ref[...])
        a = jnp.maximum(jnp.dot(z, w1a_ref[...],
                                preferred_element_type=jnp.float32) + b1a_ref[...], 0.0)
        h = jnp.dot(a, w1b_ref[...], preferred_element_type=jnp.float32) + b1b_ref[...]
        o_ref[...] = jnp.maximum(h, 0.0)

    return pl.pallas_call(
        body,
        out_shape=jax.ShapeDtypeStruct((_NP, _H), jnp.float32),
    )(x, p0, p1, W1a, b1a, W1b, b1b)


def _tc_scores(rh, p0, p1, W2a, b2a, W2b, b2b):
    def body(rh_ref, p0_ref, p1_ref, w2a_ref, b2a_ref, w2b_ref, b2b_ref, o_ref):
        z = rh_ref[...] + (p0_ref[...] + p1_ref[...])
        g = jnp.maximum(jnp.dot(z, w2a_ref[...],
                                preferred_element_type=jnp.float32) + b2a_ref[...], 0.0)
        raw = jnp.dot(g, w2b_ref[...], preferred_element_type=jnp.float32) + b2b_ref[...]
        o_ref[...] = jax.nn.sigmoid(raw)

    return pl.pallas_call(
        body,
        out_shape=jax.ShapeDtypeStruct((_NP, 1), jnp.float32),
    )(rh, p0, p1, W2a, b2a, W2b, b2b)


def _tc_keys(s2, n2, bf2):
    def body(s_ref, n_ref, b_ref, o_ref):
        v = s_ref[...] + n_ref[...]
        row = lax.broadcasted_iota(jnp.int32, (_NROW, 128), 0)
        lane = lax.broadcasted_iota(jnp.int32, (_NROW, 128), 1)
        valid = (row * 128 + lane) < _N
        mn = jnp.min(jnp.where(valid, v, jnp.inf))
        mx = jnp.max(jnp.where(valid, v, -jnp.inf))
        rng = mx - mn
        norm = v - mn
        norm = jnp.where(rng > 0, norm / (rng + 1e-5), norm)
        o_ref[...] = jnp.where(valid, b_ref[...] + norm, 1e9)

    return pl.pallas_call(
        body,
        out_shape=jax.ShapeDtypeStruct((_NROW, 128), jnp.float32),
    )(s2, n2, bf2)


def _tc_ranks_perm(keys2, lo, hi, plo, phi):
    def body(keys_ref, lo_ref, hi_ref, plo_ref, phi_ref, o_ref, ranks_s):
        pid = pl.program_id(0)
        iota_s = lax.broadcasted_iota(jnp.int32, (128, 128), 0)
        iota_l = lax.broadcasted_iota(jnp.int32, (128, 128), 1)
        eye = iota_s == iota_l

        @pl.when(pid < _NROW)
        def _():
            bi = pid
            ki_row = keys_ref[pl.ds(bi, 1), :]
            ki_col = jnp.sum(jnp.where(eye, ki_row, 0.0), axis=1, keepdims=True)
            gi = bi * 128 + lax.broadcasted_iota(jnp.int32, (128, 1), 0)

            def jbody(jc, acc):
                kj = keys_ref[pl.ds(jc, 1), :]
                gj = jc * 128 + lax.broadcasted_iota(jnp.int32, (1, 128), 1)
                lt = kj < ki_col
                tie = (kj == ki_col) & (gj < gi)
                m = jnp.where(lt | tie, 1.0, 0.0)
                return acc + jnp.sum(m, axis=1, keepdims=True)

            acc0 = jnp.full((128, 1), 128.0) * lo_ref[bi].astype(jnp.float32)
            acc = lax.fori_loop(lo_ref[bi], hi_ref[bi], jbody, acc0)
            row = jnp.sum(jnp.where(eye, acc, 0.0), axis=0, keepdims=True)
            ranks_s[pl.ds(bi, 1), :] = row.astype(jnp.int32)

        @pl.when(pid >= _NROW)
        def _():
            bp = pid - _NROW
            prow = bp * 128 + lax.broadcasted_iota(jnp.int32, (1, 128), 1)

            def ibody(bi, acc):
                rrow = ranks_s[pl.ds(bi, 1), :]
                rcol = jnp.sum(jnp.where(eye, rrow, 0), axis=1, keepdims=True)
                gcol = bi * 128 + lax.broadcasted_iota(jnp.int32, (128, 1), 0)
                match = rcol == prow
                return acc + jnp.sum(jnp.where(match, gcol, 0),
                                     axis=0, keepdims=True)

            acc = lax.fori_loop(plo_ref[bp], phi_ref[bp], ibody,
                                jnp.zeros((1, 128), jnp.int32))
            o_ref[...] = acc.reshape(1, 1, 128)

    return pl.pallas_call(
        body,
        grid=(2 * _NROW,),
        in_specs=[
            pl.BlockSpec((_NROW, 128), lambda i: (0, 0)),
            pl.BlockSpec(memory_space=pltpu.SMEM),
            pl.BlockSpec(memory_space=pltpu.SMEM),
            pl.BlockSpec(memory_space=pltpu.SMEM),
            pl.BlockSpec(memory_space=pltpu.SMEM),
        ],
        out_specs=pl.BlockSpec((1, 1, 128),
                               lambda i: (jnp.maximum(i - _NROW, 0), 0, 0)),
        out_shape=jax.ShapeDtypeStruct((_NROW, 1, 128), jnp.int32),
        scratch_shapes=[pltpu.VMEM((_NROW, 128), jnp.int32)],
    )(keys2, lo, hi, plo, phi)


def kernel(x, edge_index, batch, W1a, b1a, W1b, b1b, W2a, b2a, W2b, b2b):
    src = edge_index[0].astype(jnp.int32)
    dst = edge_index[1].astype(jnp.int32)
    srcp = jnp.concatenate([src, jnp.zeros((_EPAD - _E,), jnp.int32)])
    dstp = jnp.concatenate([dst, jnp.full((_EPAD - _E,), _NP - 1, jnp.int32)])
    idxp = jnp.stack([srcp.reshape(-1, _CHUNK), dstp.reshape(-1, _CHUNK)], axis=1)
    x_pad = jnp.pad(x, ((0, _NP - _N), (0, 0)))
    zrowsD = jnp.zeros((_NP, _D), jnp.float32)
    zrowsH = jnp.zeros((_NP, _H), jnp.float32)

    p1 = _segsum_sc(x_pad, idxp, zrowsD, _D, 120)
    rh = _tc_mid(x_pad, p1[0], p1[1], W1a, b1a.reshape(1, _H), W1b,
                 b1b.reshape(1, _H))
    p2 = _segsum_sc(rh, idxp, zrowsH, _H, 91)
    scoresA = _tc_scores(rh, p2[0], p2[1], W2a, b2a.reshape(1, _H), W2b,
                         b2b.reshape(1, 1))

    s2 = scoresA.reshape(_NROW, 128)
    noise = jax.random.uniform(jax.random.key(42), (_N,), dtype=jnp.float32) * 1e-05
    n2 = jnp.pad(noise, (0, _NP - _N)).reshape(_NROW, 128)
    batch_pad = jnp.concatenate([batch.astype(jnp.int32),
                                 jnp.full((_NP - _N,), 127, jnp.int32)])
    bf2 = batch_pad.astype(jnp.float32).reshape(_NROW, 128)
    keys2 = _tc_keys(s2, n2, bf2)

    bp2 = batch_pad.reshape(_NROW, 128)
    bmin, bmax = bp2[:, 0], bp2[:, 127]
    lo = jnp.sum((bmax[None, :] < bmin[:, None]), axis=1).astype(jnp.int32)
    hi = jnp.sum((bmin[None, :] <= bmax[:, None]), axis=1).astype(jnp.int32)
    plo = jnp.sum((hi[None, :] <= jnp.arange(_NROW)[:, None]), axis=1).astype(jnp.int32)
    phi = jnp.sum((lo[None, :] <= jnp.arange(_NROW)[:, None]), axis=1).astype(jnp.int32)
    perm3 = _tc_ranks_perm(keys2, lo, hi, plo, phi)

    perm = perm3.reshape(_NP)[:_N]
    scores = scoresA.reshape(_NP)[:_N]
    return (perm, scores)


# speedup vs baseline: 1.0934x; 1.0087x over previous
import functools

import jax
import jax.numpy as jnp
from jax import lax
from jax.experimental import pallas as pl
from jax.experimental.pallas import tpu as pltpu
from jax.experimental.pallas import tpu_sc as plsc

_N = 10000
_E = 320000
_D = 128
_H = 32
_NP = 10240
_NROW = _NP // 128
_NWORK = 32
_CHUNK = 128
_EPW = 10112
_EPAD = _NWORK * _EPW
_NCHUNK = _EPW // _CHUNK
_ROWS_PS = _NP // 16


def _segsum_sc(table, idxp, zrows, width, k0):
    mesh = plsc.VectorSubcoreMesh(core_axis_name="c", subcore_axis_name="s")
    k1 = 2 * _NCHUNK - k0

    @functools.partial(
        pl.kernel,
        out_type=jax.ShapeDtypeStruct((2, _NP, width), jnp.float32),
        mesh=mesh,
        compiler_params=pltpu.CompilerParams(use_tc_tiling_on_sc=False),
        scratch_types=[
            pltpu.VMEM((3, 2, _CHUNK), jnp.int32),
            pltpu.VMEM((2, _CHUNK, width), jnp.float32),
            pltpu.VMEM_SHARED((_NP, width), jnp.float32),
            pltpu.SemaphoreType.DMA,
            pltpu.SemaphoreType.DMA,
            pltpu.SemaphoreType.DMA,
        ],
    )
    def k(table_hbm, idx_hbm, zero_hbm, out_hbm, ib, rows, acc, sem_i, sem_g, sem_s):
        c = lax.axis_index("c")
        s = lax.axis_index("s")
        r0 = s * _ROWS_PS
        pltpu.sync_copy(zero_hbm.at[pl.ds(r0, _ROWS_PS)], acc.at[pl.ds(r0, _ROWS_PS)])
        plsc.subcore_barrier()
        bc = jnp.where(c == 0, s * k0, 16 * k0 + s * k1)
        nch = jnp.where(c == 0, k0, k1)

        pltpu.async_copy(idx_hbm.at[bc], ib.at[0], sem_i)
        pltpu.make_async_copy(idx_hbm.at[bc], ib.at[0], sem_i).wait()
        pltpu.async_copy(table_hbm.at[ib.at[0, 0]], rows.at[0], sem_g)
        pltpu.async_copy(idx_hbm.at[bc + 1], ib.at[1], sem_i)

        def body(k_, carry):
            p = lax.rem(k_, 2)
            i3 = lax.rem(k_, 3)
            pltpu.make_async_copy(table_hbm.at[ib.at[i3, 0]], rows.at[p], sem_g).wait()
            pltpu.async_copy(rows.at[p], acc.at[ib.at[i3, 1]], sem_s, add=True)

            @pl.when(k_ >= 1)
            def _():
                pn = lax.rem(k_ + 1, 2)
                i3m = lax.rem(k_ + 2, 3)
                pltpu.make_async_copy(rows.at[pn], acc.at[ib.at[i3m, 1]], sem_s).wait()

            @pl.when(k_ + 2 < nch)
            def _():
                i3p2 = lax.rem(k_ + 2, 3)
                pltpu.async_copy(idx_hbm.at[bc + k_ + 2], ib.at[i3p2], sem_i)

            @pl.when(k_ + 1 < nch)
            def _():
                pn = lax.rem(k_ + 1, 2)
                i3p = lax.rem(k_ + 1, 3)
                pltpu.make_async_copy(idx_hbm.at[bc + k_ + 1], ib.at[i3p], sem_i).wait()
                pltpu.async_copy(table_hbm.at[ib.at[i3p, 0]], rows.at[pn], sem_g)

            return carry

        lax.fori_loop(0, nch, body, 0)
        pltpu.make_async_copy(
            rows.at[lax.rem(nch - 1, 2)],
            acc.at[ib.at[lax.rem(nch - 1, 3), 1]], sem_s).wait()
        plsc.subcore_barrier()
        pltpu.sync_copy(acc.at[pl.ds(r0, _ROWS_PS)], out_hbm.at[c, pl.ds(r0, _ROWS_PS)])

    return k(table, idxp, zrows)


def _tc_mid(x, p0, p1, W1a, b1a, W1b, b1b):
    def body(x_ref, p0_ref, p1_ref, w1a_ref, b1a_ref, w1b_ref, b1b_ref, o_ref):
        z = x_ref[...] + (p0_ref[...] + p1_ref[...])
        a = jnp.maximum(jnp.dot(z, w1a_ref[...],
                                preferred_element_type=jnp.float32) + b1a_ref[...], 0.0)
        h = jnp.dot(a, w1b_ref[...], preferred_element_type=jnp.float32) + b1b_ref[...]
        o_ref[...] = jnp.maximum(h, 0.0)

    return pl.pallas_call(
        body,
        out_shape=jax.ShapeDtypeStruct((_NP, _H), jnp.float32),
    )(x, p0, p1, W1a, b1a, W1b, b1b)


def _tc_scores(rh, p0, p1, W2a, b2a, W2b, b2b):
    def body(rh_ref, p0_ref, p1_ref, w2a_ref, b2a_ref, w2b_ref, b2b_ref, o_ref):
        z = rh_ref[...] + (p0_ref[...] + p1_ref[...])
        g = jnp.maximum(jnp.dot(z, w2a_ref[...],
                                preferred_element_type=jnp.float32) + b2a_ref[...], 0.0)
        raw = jnp.dot(g, w2b_ref[...], preferred_element_type=jnp.float32) + b2b_ref[...]
        o_ref[...] = jax.nn.sigmoid(raw)

    return pl.pallas_call(
        body,
        out_shape=jax.ShapeDtypeStruct((_NP, 1), jnp.float32),
    )(rh, p0, p1, W2a, b2a, W2b, b2b)


def _tc_keys(s2, n2, bf2):
    def body(s_ref, n_ref, b_ref, o_ref):
        v = s_ref[...] + n_ref[...]
        row = lax.broadcasted_iota(jnp.int32, (_NROW, 128), 0)
        lane = lax.broadcasted_iota(jnp.int32, (_NROW, 128), 1)
        valid = (row * 128 + lane) < _N
        mn = jnp.min(jnp.where(valid, v, jnp.inf))
        mx = jnp.max(jnp.where(valid, v, -jnp.inf))
        rng = mx - mn
        norm = v - mn
        norm = jnp.where(rng > 0, norm / (rng + 1e-5), norm)
        o_ref[...] = jnp.where(valid, b_ref[...] + norm, 1e9)

    return pl.pallas_call(
        body,
        out_shape=jax.ShapeDtypeStruct((_NROW, 128), jnp.float32),
    )(s2, n2, bf2)


def _tc_ranks_perm(keys2, lo, hi, plo, phi):
    def body(keys_ref, lo_ref, hi_ref, plo_ref, phi_ref, o_ref, ranks_s):
        pid = pl.program_id(0)
        iota_s = lax.broadcasted_iota(jnp.int32, (128, 128), 0)
        iota_l = lax.broadcasted_iota(jnp.int32, (128, 128), 1)
        eye = iota_s == iota_l

        @pl.when(pid < _NROW)
        def _():
            bi = pid
            ki_row = keys_ref[pl.ds(bi, 1), :]
            ki_col = jnp.sum(jnp.where(eye, ki_row, 0.0), axis=1, keepdims=True)
            gi = bi * 128 + lax.broadcasted_iota(jnp.int32, (128, 1), 0)

            def jbody(jc, acc):
                kj = keys_ref[pl.ds(jc, 1), :]
                gj = jc * 128 + lax.broadcasted_iota(jnp.int32, (1, 128), 1)
                lt = kj < ki_col
                tie = (kj == ki_col) & (gj < gi)
                m = jnp.where(lt | tie, 1.0, 0.0)
                return acc + jnp.sum(m, axis=1, keepdims=True)

            acc0 = jnp.full((128, 1), 128.0) * lo_ref[bi].astype(jnp.float32)
            acc = lax.fori_loop(lo_ref[bi], hi_ref[bi], jbody, acc0)
            row = jnp.sum(jnp.where(eye, acc, 0.0), axis=0, keepdims=True)
            ranks_s[pl.ds(bi, 1), :] = row.astype(jnp.int32)

        @pl.when(pid >= _NROW)
        def _():
            bp = pid - _NROW
            prow = bp * 128 + lax.broadcasted_iota(jnp.int32, (1, 128), 1)

            def ibody(bi, acc):
                rrow = ranks_s[pl.ds(bi, 1), :]
                rcol = jnp.sum(jnp.where(eye, rrow, 0), axis=1, keepdims=True)
                gcol = bi * 128 + lax.broadcasted_iota(jnp.int32, (128, 1), 0)
                match = rcol == prow
                return acc + jnp.sum(jnp.where(match, gcol, 0),
                                     axis=0, keepdims=True)

            acc = lax.fori_loop(plo_ref[bp], phi_ref[bp], ibody,
                                jnp.zeros((1, 128), jnp.int32))
            o_ref[...] = acc.reshape(1, 1, 128)

    return pl.pallas_call(
        body,
        grid=(2 * _NROW,),
        in_specs=[
            pl.BlockSpec((_NROW, 128), lambda i: (0, 0)),
            pl.BlockSpec(memory_space=pltpu.SMEM),
            pl.BlockSpec(memory_space=pltpu.SMEM),
            pl.BlockSpec(memory_space=pltpu.SMEM),
            pl.BlockSpec(memory_space=pltpu.SMEM),
        ],
        out_specs=pl.BlockSpec((1, 1, 128),
                               lambda i: (jnp.maximum(i - _NROW, 0), 0, 0)),
        out_shape=jax.ShapeDtypeStruct((_NROW, 1, 128), jnp.int32),
        scratch_shapes=[pltpu.VMEM((_NROW, 128), jnp.int32)],
    )(keys2, lo, hi, plo, phi)


def kernel(x, edge_index, batch, W1a, b1a, W1b, b1b, W2a, b2a, W2b, b2b):
    src = edge_index[0].astype(jnp.int32)
    dst = edge_index[1].astype(jnp.int32)
    srcp = jnp.concatenate([src, jnp.zeros((_EPAD - _E,), jnp.int32)])
    dstp = jnp.concatenate([dst, jnp.full((_EPAD - _E,), _NP - 1, jnp.int32)])
    idxp = jnp.stack([srcp.reshape(-1, _CHUNK), dstp.reshape(-1, _CHUNK)], axis=1)
    x_pad = jnp.pad(x, ((0, _NP - _N), (0, 0)))
    zrowsD = jnp.zeros((_NP, _D), jnp.float32)
    zrowsH = jnp.zeros((_NP, _H), jnp.float32)

    p1 = _segsum_sc(x_pad, idxp, zrowsD, _D, 127)
    rh = _tc_mid(x_pad, p1[0], p1[1], W1a, b1a.reshape(1, _H), W1b,
                 b1b.reshape(1, _H))
    p2 = _segsum_sc(rh, idxp, zrowsH, _H, 99)
    scoresA = _tc_scores(rh, p2[0], p2[1], W2a, b2a.reshape(1, _H), W2b,
                         b2b.reshape(1, 1))

    s2 = scoresA.reshape(_NROW, 128)
    noise = jax.random.uniform(jax.random.key(42), (_N,), dtype=jnp.float32) * 1e-05
    n2 = jnp.pad(noise, (0, _NP - _N)).reshape(_NROW, 128)
    batch_pad = jnp.concatenate([batch.astype(jnp.int32),
                                 jnp.full((_NP - _N,), 127, jnp.int32)])
    bf2 = batch_pad.astype(jnp.float32).reshape(_NROW, 128)
    keys2 = _tc_keys(s2, n2, bf2)

    bp2 = batch_pad.reshape(_NROW, 128)
    bmin, bmax = bp2[:, 0], bp2[:, 127]
    lo = jnp.sum((bmax[None, :] < bmin[:, None]), axis=1).astype(jnp.int32)
    hi = jnp.sum((bmin[None, :] <= bmax[:, None]), axis=1).astype(jnp.int32)
    plo = jnp.sum((hi[None, :] <= jnp.arange(_NROW)[:, None]), axis=1).astype(jnp.int32)
    phi = jnp.sum((lo[None, :] <= jnp.arange(_NROW)[:, None]), axis=1).astype(jnp.int32)
    perm3 = _tc_ranks_perm(keys2, lo, hi, plo, phi)

    perm = perm3.reshape(_NP)[:_N]
    scores = scoresA.reshape(_NP)[:_N]
    return (perm, scores)
